# Initial kernel scaffold; baseline (speedup 1.0000x reference)
import jax
import jax.numpy as jnp
from jax.experimental import pallas as pl


def kernel(x, pe):
    raise NotImplementedError("write your pallas kernel here")


# speedup vs baseline: 4.9925x; 4.9925x over previous
import functools

import jax
import jax.numpy as jnp
from jax import lax
from jax.experimental import pallas as pl
from jax.experimental.pallas import tpu as pltpu
from jax.experimental.pallas import tpu_sc as plsc

DIM = 64
ROW = 128
NROWS = 16384 * 200 // ROW
K = 8


@functools.lru_cache(maxsize=None)
def _build():
    info = plsc.get_sparse_core_info()
    nw = info.num_cores * info.num_subcores
    rows_per_w = NROWS // nw
    n_iters = rows_per_w // K
    mesh = plsc.VectorSubcoreMesh(core_axis_name="c", subcore_axis_name="s")

    @functools.partial(
        pl.kernel,
        mesh=mesh,
        out_type=jax.ShapeDtypeStruct((NROWS, ROW, DIM), jnp.float32),
        scratch_types=[
            pltpu.VMEM((K, ROW), jnp.int32),
            pltpu.VMEM((K, ROW, DIM), jnp.float32),
            pltpu.SemaphoreType.DMA,
        ],
        compiler_params=pltpu.CompilerParams(use_tc_tiling_on_sc=False),
    )
    def gather_kernel(idx_hbm, table_hbm, out_hbm, idx_v, rows_v, sem):
        wid = lax.axis_index("s") * info.num_cores + lax.axis_index("c")
        base = wid * rows_per_w

        def body(i, carry):
            r0 = base + i * K
            pltpu.sync_copy(idx_hbm.at[pl.ds(r0, K)], idx_v)
            copies = [
                pltpu.async_copy(table_hbm.at[idx_v.at[j]], rows_v.at[j], sem)
                for j in range(K)
            ]
            for c in copies:
                c.wait()
            pltpu.sync_copy(rows_v, out_hbm.at[pl.ds(r0, K)])
            return carry

        lax.fori_loop(0, n_iters, body, 0)

    return gather_kernel


def kernel(x, pe):
    idx = x.astype(jnp.int32).reshape(NROWS, ROW)
    out = _build()(idx, pe)
    return out.reshape(x.shape[0], x.shape[1], DIM)


# speedup vs baseline: 5.1611x; 1.0338x over previous
import functools

import jax
import jax.numpy as jnp
from jax import lax
from jax.experimental import pallas as pl
from jax.experimental.pallas import tpu as pltpu
from jax.experimental.pallas import tpu_sc as plsc

DIM = 64
ROW = 128
NROWS = 16384 * 200 // ROW
K = 4
NIB = 4
NRB = 2


@functools.lru_cache(maxsize=None)
def _build():
    info = plsc.get_sparse_core_info()
    nw = info.num_cores * info.num_subcores
    rows_per_w = NROWS // nw
    chunks = rows_per_w // K
    n_outer = chunks // NIB
    mesh = plsc.VectorSubcoreMesh(core_axis_name="c", subcore_axis_name="s")

    @functools.partial(
        pl.kernel,
        mesh=mesh,
        out_type=jax.ShapeDtypeStruct((NROWS, ROW, DIM), jnp.float32),
        scratch_types=[
            pltpu.VMEM((NIB, K, ROW), jnp.int32),
            pltpu.VMEM((NRB, K, ROW, DIM), jnp.float32),
            [pltpu.SemaphoreType.DMA] * NIB,
            [pltpu.SemaphoreType.DMA] * NRB,
            [pltpu.SemaphoreType.DMA] * NRB,
        ],
        compiler_params=pltpu.CompilerParams(use_tc_tiling_on_sc=False),
    )
    def gather_kernel(idx_hbm, table_hbm, out_hbm, idx_v, rows_v,
                      isems, gsems, ssems):
        wid = lax.axis_index("s") * info.num_cores + lax.axis_index("c")
        base = wid * rows_per_w

        def fire_idx(c, ib):
            r0 = base + c * K
            pltpu.async_copy(idx_hbm.at[pl.ds(r0, K)], idx_v.at[ib],
                             isems[ib])

        def wait_idx(c, ib):
            r0 = base + c * K
            pltpu.make_async_copy(idx_hbm.at[pl.ds(r0, K)], idx_v.at[ib],
                                  isems[ib]).wait()

        def fire_gathers(ib, rb):
            for j in range(K):
                pltpu.async_copy(table_hbm.at[idx_v.at[ib, j]],
                                 rows_v.at[rb, j], gsems[rb])

        def wait_gathers(ib, rb):
            for j in range(K):
                pltpu.make_async_copy(table_hbm.at[idx_v.at[ib, j]],
                                      rows_v.at[rb, j], gsems[rb]).wait()

        def fire_store(c, rb):
            r0 = base + c * K
            pltpu.async_copy(rows_v.at[rb], out_hbm.at[pl.ds(r0, K)],
                             ssems[rb])

        def wait_store(c, rb):
            r0 = base + c * K
            pltpu.make_async_copy(rows_v.at[rb], out_hbm.at[pl.ds(r0, K)],
                                  ssems[rb]).wait()

        pltpu.sync_copy(idx_hbm.at[pl.ds(base, K)], idx_v.at[0])

        def body(t, carry):
            for b in range(NIB):
                c = t * NIB + b
                rb, rbp = b % NRB, (b - 1) % NRB

                @pl.when(c + 1 < chunks)
                def _():
                    fire_idx(c + 1, (b + 1) % NIB)

                @pl.when(c - NRB >= 0)
                def _():
                    wait_store(c - NRB, rb)

                @pl.when(c >= 1)
                def _():
                    wait_idx(c, b)

                fire_gathers(b, rb)

                @pl.when(c >= 1)
                def _():
                    wait_gathers((b - 1) % NIB, rbp)
                    fire_store(c - 1, rbp)

            return carry

        lax.fori_loop(0, n_outer, body, 0)

        last = chunks - 1
        wait_gathers((NIB - 1) % NIB, (NIB - 1) % NRB)
        fire_store(last, (NIB - 1) % NRB)
        wait_store(last - 1, (NIB - 2) % NRB)
        wait_store(last, (NIB - 1) % NRB)

    return gather_kernel


def kernel(x, pe):
    idx = x.astype(jnp.int32).reshape(NROWS, ROW)
    out = _build()(idx, pe)
    return out.reshape(x.shape[0], x.shape[1], DIM)
